# baseline (speedup 1.0000x reference)
import jax
import jax.numpy as jnp
from jax.experimental import pallas as pl
from jax.experimental.pallas import tpu as pltpu

_N = 10000
_IN = 128
_OUT = 128
_T = 4
_C = 10
_R = _N // _C


def _body(t_hbm, x_hbm, w_hbm, b_hbm, o_hbm,
          t_v, x_v, wc_v, b_v, o_v,
          in_sems, aux_sem, out_sems):
    aux_copies = [
        pltpu.make_async_copy(b_hbm, b_v, aux_sem.at[0]),
        pltpu.make_async_copy(t_hbm, t_v, aux_sem.at[1]),
    ]
    for t in range(_T):
        aux_copies.append(pltpu.make_async_copy(
            w_hbm.at[t], wc_v.at[:, t * _OUT:(t + 1) * _OUT], aux_sem.at[2 + t]))
    for c in aux_copies:
        c.start()
    in_copies = []
    for i in range(_C):
        sl = pl.ds(i * _R, _R)
        c = pltpu.make_async_copy(x_hbm.at[sl, :], x_v.at[sl, :], in_sems.at[i])
        c.start(priority=i % 2)
        in_copies.append(c)
    for c in aux_copies:
        c.wait()

    wcb = wc_v[...].astype(jnp.bfloat16)
    out_copies = []
    for i in range(_C):
        sl = pl.ds(i * _R, _R)
        in_copies[i].wait()
        xv = x_v[sl, :].astype(jnp.bfloat16)
        tv = t_v[sl].reshape(_R, 1)
        y = jnp.dot(xv, wcb, preferred_element_type=jnp.float32)
        out = y[:, 3 * _OUT:]
        bias = b_v[3]
        for t in (2, 1, 0):
            sel = tv == t
            out = jnp.where(sel, y[:, t * _OUT:(t + 1) * _OUT], out)
            bias = jnp.where(sel, b_v[t], bias)
        o_v[sl, :] = out + bias
        c = pltpu.make_async_copy(o_v.at[sl, :], o_hbm.at[sl, :], out_sems.at[i])
        c.start(priority=i % 2)
        out_copies.append(c)
    for c in out_copies:
        c.wait()


def kernel(x, edge_index, node_types, W, b):
    del edge_index
    return pl.pallas_call(
        _body,
        in_specs=[
            pl.BlockSpec(memory_space=pl.ANY),
            pl.BlockSpec(memory_space=pl.ANY),
            pl.BlockSpec(memory_space=pl.ANY),
            pl.BlockSpec(memory_space=pl.ANY),
        ],
        out_specs=pl.BlockSpec(memory_space=pl.ANY),
        out_shape=jax.ShapeDtypeStruct((_N, _OUT), jnp.float32),
        scratch_shapes=[
            pltpu.VMEM((_N,), jnp.int32),
            pltpu.VMEM((_N, _IN), jnp.float32),
            pltpu.VMEM((_IN, _T * _OUT), jnp.float32),
            pltpu.VMEM((_T, _OUT), jnp.float32),
            pltpu.VMEM((_N, _OUT), jnp.float32),
            pltpu.SemaphoreType.DMA((_C,)),
            pltpu.SemaphoreType.DMA((2 + _T,)),
            pltpu.SemaphoreType.DMA((_C,)),
        ],
    )(node_types, x, W, b)


# speedup vs baseline: 1.0025x; 1.0025x over previous
import jax
import jax.numpy as jnp
from jax.experimental import pallas as pl
from jax.experimental.pallas import tpu as pltpu

_N = 10000
_IN = 128
_OUT = 128
_T = 4
_C = 10
_R = _N // _C


def _body(t_hbm, x_hbm, x_hbm2, w_hbm, b_hbm, o_hbm,
          t_v, x_v, wc_v, b_v, o_v,
          in_sems, aux_sem, out_sems):
    x_refs = (x_hbm, x_hbm2)
    aux_copies = [
        pltpu.make_async_copy(b_hbm, b_v, aux_sem.at[0]),
        pltpu.make_async_copy(t_hbm, t_v, aux_sem.at[1]),
    ]
    for t in range(_T):
        aux_copies.append(pltpu.make_async_copy(
            w_hbm.at[t], wc_v.at[:, t * _OUT:(t + 1) * _OUT], aux_sem.at[2 + t]))
    for c in aux_copies:
        c.start()
    in_copies = []
    for i in range(_C):
        sl = pl.ds(i * _R, _R)
        c = pltpu.make_async_copy(
            x_refs[(i // 2) % 2].at[sl, :], x_v.at[sl, :], in_sems.at[i])
        c.start(priority=i % 2)
        in_copies.append(c)
    for c in aux_copies:
        c.wait()

    wcb = wc_v[...].astype(jnp.bfloat16)
    out_copies = []
    for i in range(_C):
        sl = pl.ds(i * _R, _R)
        in_copies[i].wait()
        xv = x_v[sl, :].astype(jnp.bfloat16)
        tv = t_v[sl].reshape(_R, 1)
        y = jnp.dot(xv, wcb, preferred_element_type=jnp.float32)
        out = y[:, 3 * _OUT:]
        bias = b_v[3]
        for t in (2, 1, 0):
            sel = tv == t
            out = jnp.where(sel, y[:, t * _OUT:(t + 1) * _OUT], out)
            bias = jnp.where(sel, b_v[t], bias)
        o_v[sl, :] = out + bias
        c = pltpu.make_async_copy(o_v.at[sl, :], o_hbm.at[sl, :], out_sems.at[i])
        c.start()
        out_copies.append(c)
    for c in out_copies:
        c.wait()


def kernel(x, edge_index, node_types, W, b):
    del edge_index
    return pl.pallas_call(
        _body,
        in_specs=[
            pl.BlockSpec(memory_space=pl.ANY),
            pl.BlockSpec(memory_space=pl.ANY),
            pl.BlockSpec(memory_space=pl.ANY),
            pl.BlockSpec(memory_space=pl.ANY),
            pl.BlockSpec(memory_space=pl.ANY),
        ],
        out_specs=pl.BlockSpec(memory_space=pl.ANY),
        out_shape=jax.ShapeDtypeStruct((_N, _OUT), jnp.float32),
        scratch_shapes=[
            pltpu.VMEM((_N,), jnp.int32),
            pltpu.VMEM((_N, _IN), jnp.float32),
            pltpu.VMEM((_IN, _T * _OUT), jnp.float32),
            pltpu.VMEM((_T, _OUT), jnp.float32),
            pltpu.VMEM((_N, _OUT), jnp.float32),
            pltpu.SemaphoreType.DMA((_C,)),
            pltpu.SemaphoreType.DMA((2 + _T,)),
            pltpu.SemaphoreType.DMA((_C,)),
        ],
    )(node_types, x, x, W, b)


# speedup vs baseline: 1.1471x; 1.1442x over previous
import jax
import jax.numpy as jnp
from jax.experimental import pallas as pl
from jax.experimental.pallas import tpu as pltpu

_N = 10000
_IN = 128
_OUT = 128
_T = 4
_C = 5
_R = _N // _C


def _body(t_hbm, x_hbm, x_hbm2, w_hbm, b_hbm, o_hbm,
          t_v, x_v, wc_v, b_v, o_v,
          in_sems, aux_sem, out_sems):
    x_refs = (x_hbm, x_hbm2)
    aux_copies = [
        pltpu.make_async_copy(b_hbm, b_v, aux_sem.at[0]),
        pltpu.make_async_copy(t_hbm, t_v, aux_sem.at[1]),
    ]
    for t in range(_T):
        aux_copies.append(pltpu.make_async_copy(
            w_hbm.at[t], wc_v.at[:, t * _OUT:(t + 1) * _OUT], aux_sem.at[2 + t]))
    for c in aux_copies:
        c.start()
    in_copies = []
    for i in range(_C):
        sl = pl.ds(i * _R, _R)
        c = pltpu.make_async_copy(
            x_refs[(i // 2) % 2].at[sl, :], x_v.at[sl, :], in_sems.at[i])
        c.start(priority=i % 2)
        in_copies.append(c)
    for c in aux_copies:
        c.wait()

    wcb = wc_v[...].astype(jnp.bfloat16)
    out_copies = []
    for i in range(_C):
        sl = pl.ds(i * _R, _R)
        in_copies[i].wait()
        xv = x_v[sl, :].astype(jnp.bfloat16)
        tv = t_v[sl].reshape(_R, 1)
        y = jnp.dot(xv, wcb, preferred_element_type=jnp.float32)
        out = y[:, 3 * _OUT:]
        bias = b_v[3]
        for t in (2, 1, 0):
            sel = tv == t
            out = jnp.where(sel, y[:, t * _OUT:(t + 1) * _OUT], out)
            bias = jnp.where(sel, b_v[t], bias)
        o_v[sl, :] = out + bias
        c = pltpu.make_async_copy(o_v.at[sl, :], o_hbm.at[sl, :], out_sems.at[i])
        c.start()
        out_copies.append(c)
    for c in out_copies:
        c.wait()


def kernel(x, edge_index, node_types, W, b):
    del edge_index
    return pl.pallas_call(
        _body,
        in_specs=[
            pl.BlockSpec(memory_space=pl.ANY),
            pl.BlockSpec(memory_space=pl.ANY),
            pl.BlockSpec(memory_space=pl.ANY),
            pl.BlockSpec(memory_space=pl.ANY),
            pl.BlockSpec(memory_space=pl.ANY),
        ],
        out_specs=pl.BlockSpec(memory_space=pl.ANY),
        out_shape=jax.ShapeDtypeStruct((_N, _OUT), jnp.float32),
        scratch_shapes=[
            pltpu.VMEM((_N,), jnp.int32),
            pltpu.VMEM((_N, _IN), jnp.float32),
            pltpu.VMEM((_IN, _T * _OUT), jnp.float32),
            pltpu.VMEM((_T, _OUT), jnp.float32),
            pltpu.VMEM((_N, _OUT), jnp.float32),
            pltpu.SemaphoreType.DMA((_C,)),
            pltpu.SemaphoreType.DMA((2 + _T,)),
            pltpu.SemaphoreType.DMA((_C,)),
        ],
    )(node_types, x, x, W, b)


# speedup vs baseline: 1.1523x; 1.0045x over previous
import jax
import jax.numpy as jnp
from jax.experimental import pallas as pl
from jax.experimental.pallas import tpu as pltpu

_N = 10000
_IN = 128
_OUT = 128
_T = 4
_C = 5
_R = _N // _C


def _body(t_hbm, x_hbm, x_hbm2, w_hbm, b_hbm, o_hbm,
          t_v, x_v, wc_v, b_v, o_v,
          in_sems, aux_sem, out_sems):
    x_refs = (x_hbm, x_hbm2)
    aux_copies = [
        pltpu.make_async_copy(b_hbm, b_v, aux_sem.at[0]),
        pltpu.make_async_copy(t_hbm, t_v, aux_sem.at[1]),
    ]
    for t in range(_T):
        aux_copies.append(pltpu.make_async_copy(
            w_hbm.at[t], wc_v.at[:, t * _OUT:(t + 1) * _OUT], aux_sem.at[2 + t]))
    in_copies = []
    for i in range(_C):
        sl = pl.ds(i * _R, _R)
        c = pltpu.make_async_copy(
            x_refs[(i // 2) % 2].at[sl, :], x_v.at[sl, :], in_sems.at[i])
        in_copies.append(c)
    in_copies[0].start(priority=0)
    in_copies[1].start(priority=1)
    for c in aux_copies:
        c.start()
    for i in range(2, _C):
        in_copies[i].start(priority=i % 2)
    for c in aux_copies:
        c.wait()

    wcb = wc_v[...].astype(jnp.bfloat16)
    out_copies = []
    for i in range(_C):
        sl = pl.ds(i * _R, _R)
        in_copies[i].wait()
        xv = x_v[sl, :].astype(jnp.bfloat16)
        tv = t_v[sl].reshape(_R, 1)
        y = jnp.dot(xv, wcb, preferred_element_type=jnp.float32)
        out = y[:, 3 * _OUT:]
        bias = b_v[3]
        for t in (2, 1, 0):
            sel = tv == t
            out = jnp.where(sel, y[:, t * _OUT:(t + 1) * _OUT], out)
            bias = jnp.where(sel, b_v[t], bias)
        o_v[sl, :] = out + bias
        c = pltpu.make_async_copy(o_v.at[sl, :], o_hbm.at[sl, :], out_sems.at[i])
        c.start()
        out_copies.append(c)
    for c in out_copies:
        c.wait()


def kernel(x, edge_index, node_types, W, b):
    del edge_index
    return pl.pallas_call(
        _body,
        in_specs=[
            pl.BlockSpec(memory_space=pl.ANY),
            pl.BlockSpec(memory_space=pl.ANY),
            pl.BlockSpec(memory_space=pl.ANY),
            pl.BlockSpec(memory_space=pl.ANY),
            pl.BlockSpec(memory_space=pl.ANY),
        ],
        out_specs=pl.BlockSpec(memory_space=pl.ANY),
        out_shape=jax.ShapeDtypeStruct((_N, _OUT), jnp.float32),
        scratch_shapes=[
            pltpu.VMEM((_N,), jnp.int32),
            pltpu.VMEM((_N, _IN), jnp.float32),
            pltpu.VMEM((_IN, _T * _OUT), jnp.float32),
            pltpu.VMEM((_T, _OUT), jnp.float32),
            pltpu.VMEM((_N, _OUT), jnp.float32),
            pltpu.SemaphoreType.DMA((_C,)),
            pltpu.SemaphoreType.DMA((2 + _T,)),
            pltpu.SemaphoreType.DMA((_C,)),
        ],
    )(node_types, x, x, W, b)
